# baseline (speedup 1.0000x reference)
import jax
import jax.numpy as jnp
from jax.experimental import pallas as pl

_EPS = 1e-8
_ROWS = 100000
_D = 768
_BLK = 4096
_GRID = (_ROWS + _BLK - 1) // _BLK


def _body(u_ref, x_ref, o_ref):
    i = pl.program_id(0)
    u = u_ref[0, :]
    x = x_ref[...]
    dot = jnp.sum(x * u[None, :], axis=1)
    nrm = jnp.sum(x * x, axis=1)
    nu = jnp.sqrt(jnp.sum(u * u))
    denom = jnp.maximum(nu, _EPS) * jnp.maximum(jnp.sqrt(nrm), _EPS)
    o_ref[pl.ds(i, 1), :] = (dot / denom).reshape(1, _BLK)


def kernel(x, user_embed):
    out = pl.pallas_call(
        _body,
        grid=(_GRID,),
        in_specs=[
            pl.BlockSpec((1, _D), lambda i: (0, 0)),
            pl.BlockSpec((_BLK, _D), lambda i: (i, 0)),
        ],
        out_specs=pl.BlockSpec((_GRID, _BLK), lambda i: (0, 0)),
        out_shape=jax.ShapeDtypeStruct((_GRID, _BLK), jnp.float32),
    )(user_embed, x)
    return out.reshape(-1)[:_ROWS]
